# baseline (speedup 1.0000x reference)
import functools

import jax
import jax.numpy as jnp
from jax import lax
from jax.experimental import pallas as pl
from jax.experimental.pallas import tpu as pltpu
from jax.experimental.pallas import tpu_sc as plsc

_B = 16384
_H = 200
_V = 1000000
_D = 64
_C = 2

_NC = 2
_NS = 16
_NW = _NC * _NS

_IDX_PER_STREAM = 128
_NSTREAM = (_B * _H) // _IDX_PER_STREAM
_STREAMS_PER_TILE = _NSTREAM // _NW
_K = 16
_CHUNKS = _STREAMS_PER_TILE // _K


def _project_body(emb_ref, w_ref, out_ref):
    e = emb_ref[...]
    w = w_ref[...]
    p = lax.dot_general(e, w, (((1,), (1,)), ((), ())),
                        preferred_element_type=jnp.float32)
    out_ref[...] = p * (1.0 / _H)


def _project(emb, w):
    blk = 8000
    grid = _V // blk
    return pl.pallas_call(
        _project_body,
        grid=(grid,),
        in_specs=[
            pl.BlockSpec((blk, _D), lambda i: (i, 0)),
            pl.BlockSpec((_C, _D), lambda i: (0, 0)),
        ],
        out_specs=pl.BlockSpec((blk, _C), lambda i: (i, 0)),
        out_shape=jax.ShapeDtypeStruct((_V, _C), jnp.float32),
    )(emb, w)


def _gather_body(xr_hbm, p_hbm, out_hbm, idx_v, rows_v, sem):
    wid = lax.axis_index("s") * _NC + lax.axis_index("c")

    def chunk(t, carry):
        s0 = wid * _STREAMS_PER_TILE + t * _K
        pltpu.sync_copy(xr_hbm.at[pl.ds(s0, _K)], idx_v)
        cps = [
            pltpu.async_copy(p_hbm.at[idx_v.at[k]], rows_v.at[k], sem)
            for k in range(_K)
        ]
        for cp in cps:
            cp.wait()
        pltpu.sync_copy(rows_v, out_hbm.at[pl.ds(s0, _K)])
        return carry

    lax.fori_loop(0, _CHUNKS, chunk, 0)


def _gather(xr, p):
    mesh = plsc.VectorSubcoreMesh(core_axis_name="c", subcore_axis_name="s")
    kern = functools.partial(
        pl.kernel,
        mesh=mesh,
        out_type=jax.ShapeDtypeStruct((_NSTREAM, _IDX_PER_STREAM, _C),
                                      jnp.float32),
        scratch_types=[
            pltpu.VMEM((_K, _IDX_PER_STREAM), jnp.int32),
            pltpu.VMEM((_K, _IDX_PER_STREAM, _C), jnp.float32),
            pltpu.SemaphoreType.DMA,
        ],
        compiler_params=pltpu.CompilerParams(use_tc_tiling_on_sc=False),
    )(_gather_body)
    return kern(xr, p)


def _head_body(g_ref, b_ref, out_ref):
    g = g_ref[...]
    col = lax.broadcasted_iota(jnp.int32, g.shape, 1)
    s0 = jnp.sum(jnp.where(col % 2 == 0, g, 0.0), axis=1, keepdims=True)
    s1 = jnp.sum(g, axis=1, keepdims=True) - s0
    c2 = lax.broadcasted_iota(jnp.int32, (g.shape[0], _C), 1)
    logits = jnp.where(c2 == 0, s0, s1) + b_ref[...]
    m = jnp.max(logits, axis=1, keepdims=True)
    lse = m + jnp.log(jnp.sum(jnp.exp(logits - m), axis=1, keepdims=True))
    out_ref[...] = logits - lse


def _head(g, b2):
    blk = 2048
    grid = _B // blk
    return pl.pallas_call(
        _head_body,
        grid=(grid,),
        in_specs=[
            pl.BlockSpec((blk, 2 * _H), lambda i: (i, 0)),
            pl.BlockSpec((1, _C), lambda i: (0, 0)),
        ],
        out_specs=pl.BlockSpec((blk, _C), lambda i: (i, 0)),
        out_shape=jax.ShapeDtypeStruct((_B, _C), jnp.float32),
    )(g, b2)


def kernel(x, emb_table, W, b):
    p = _project(emb_table, W)
    xr = x.reshape(_NSTREAM, _IDX_PER_STREAM)
    g = _gather(xr, p)
    return _head(g.reshape(_B, 2 * _H), b.reshape(1, _C))


# speedup vs baseline: 3.6456x; 3.6456x over previous
import functools

import jax
import jax.numpy as jnp
from jax import lax
from jax.experimental import pallas as pl
from jax.experimental.pallas import tpu as pltpu
from jax.experimental.pallas import tpu_sc as plsc

_B = 16384
_H = 200
_V = 1000000
_D = 64
_C = 2

_NC = 2
_NS = 16
_NW = _NC * _NS

_IDX_PER_STREAM = 128
_NSTREAM = (_B * _H) // _IDX_PER_STREAM
_STREAMS_PER_TILE = _NSTREAM // _NW
_K = 16
_CHUNKS = _STREAMS_PER_TILE // _K

_STAGE = 62504
_VPAD = _NS * _STAGE


def _project_body(emb_ref, w_ref, out_ref):
    e = emb_ref[...]
    w = w_ref[...]
    dw = w[0:1, :] - w[1:2, :]
    q = lax.dot_general(e, dw, (((1,), (1,)), ((), ())),
                        preferred_element_type=jnp.float32)
    out_ref[...] = q * (1.0 / _H)


def _project(emb, w):
    blk = 8000
    grid = _V // blk
    return pl.pallas_call(
        _project_body,
        grid=(grid,),
        in_specs=[
            pl.BlockSpec((blk, _D), lambda i: (i, 0)),
            pl.BlockSpec((_C, _D), lambda i: (0, 0)),
        ],
        out_specs=pl.BlockSpec((blk, 1), lambda i: (i, 0)),
        out_shape=jax.ShapeDtypeStruct((_V, 1), jnp.float32),
    )(emb, w)


def _gather_body(xr_hbm, q_hbm, out_hbm,
                 q_sh, idx_a, idx_b, rows_a, rows_b,
                 stage_sem, isem_a, isem_b, gsem_a, gsem_b, ssem_a, ssem_b):
    sid = lax.axis_index("s")
    wid = sid * _NC + lax.axis_index("c")
    base = wid * _STREAMS_PER_TILE

    st0 = sid * _STAGE
    pltpu.async_copy(q_hbm.at[pl.ds(st0, _STAGE)],
                     q_sh.at[pl.ds(st0, _STAGE)], stage_sem).wait()
    plsc.subcore_barrier()

    def idx_slice(t):
        return xr_hbm.at[pl.ds(base + t * _K, _K)]

    def out_slice(t):
        return out_hbm.at[pl.ds(base + t * _K, _K)]

    pltpu.async_copy(idx_slice(0), idx_a, isem_a)
    pltpu.async_copy(idx_slice(1), idx_b, isem_b)

    def half_step(t, last_t, idx_v, rows_v, isem, gsem, ssem):
        pltpu.make_async_copy(idx_slice(t), idx_v, isem).wait()

        @pl.when(t >= 2)
        def _():
            pltpu.make_async_copy(rows_v, out_slice(t - 2), ssem).wait()

        cps = [
            pltpu.async_copy(q_sh.at[idx_v.at[k]], rows_v.at[k], gsem)
            for k in range(_K)
        ]
        for cp in cps:
            cp.wait()

        @pl.when(t + 2 <= last_t)
        def _():
            pltpu.async_copy(idx_slice(t + 2), idx_v, isem)

        pltpu.async_copy(rows_v, out_slice(t), ssem)

    def dstep(m, carry):
        half_step(2 * m, _CHUNKS - 1, idx_a, rows_a, isem_a, gsem_a, ssem_a)
        half_step(2 * m + 1, _CHUNKS - 1, idx_b, rows_b, isem_b, gsem_b, ssem_b)
        return carry

    lax.fori_loop(0, _CHUNKS // 2, dstep, 0)

    pltpu.make_async_copy(rows_a, out_slice(_CHUNKS - 2), ssem_a).wait()
    pltpu.make_async_copy(rows_b, out_slice(_CHUNKS - 1), ssem_b).wait()


def _gather(xr, qp):
    mesh = plsc.VectorSubcoreMesh(core_axis_name="c", subcore_axis_name="s")
    kern = functools.partial(
        pl.kernel,
        mesh=mesh,
        out_type=jax.ShapeDtypeStruct((_NSTREAM, _IDX_PER_STREAM),
                                      jnp.float32),
        scratch_types=[
            pltpu.VMEM_SHARED((_VPAD,), jnp.float32),
            pltpu.VMEM((_K, _IDX_PER_STREAM), jnp.int32),
            pltpu.VMEM((_K, _IDX_PER_STREAM), jnp.int32),
            pltpu.VMEM((_K, _IDX_PER_STREAM), jnp.float32),
            pltpu.VMEM((_K, _IDX_PER_STREAM), jnp.float32),
            pltpu.SemaphoreType.DMA,
            pltpu.SemaphoreType.DMA,
            pltpu.SemaphoreType.DMA,
            pltpu.SemaphoreType.DMA,
            pltpu.SemaphoreType.DMA,
            pltpu.SemaphoreType.DMA,
            pltpu.SemaphoreType.DMA,
        ],
        compiler_params=pltpu.CompilerParams(use_tc_tiling_on_sc=False),
    )(_gather_body)
    return kern(xr, qp)


def _head_body(g_ref, b_ref, out_ref):
    g = g_ref[...]
    bb = b_ref[...]
    d = jnp.sum(g, axis=1, keepdims=True) + (bb[0, 0] - bb[0, 1])
    c2 = lax.broadcasted_iota(jnp.int32, (g.shape[0], _C), 1)
    z = jnp.where(c2 == 0, -d, d)
    sp = jnp.maximum(z, 0.0) + jnp.log1p(jnp.exp(-jnp.abs(z)))
    out_ref[...] = -sp


def _head(g, b2):
    blk = 2048
    grid = _B // blk
    return pl.pallas_call(
        _head_body,
        grid=(grid,),
        in_specs=[
            pl.BlockSpec((blk, _H), lambda i: (i, 0)),
            pl.BlockSpec((1, _C), lambda i: (0, 0)),
        ],
        out_specs=pl.BlockSpec((blk, _C), lambda i: (i, 0)),
        out_shape=jax.ShapeDtypeStruct((_B, _C), jnp.float32),
    )(g, b2)


def kernel(x, emb_table, W, b):
    q = _project(emb_table, W).reshape(_V)
    qp = jnp.pad(q, (0, _VPAD - _V))
    xr = x.reshape(_NSTREAM, _IDX_PER_STREAM)
    g = _gather(xr, qp)
    return _head(g.reshape(_B, _H), b.reshape(1, _C))
